# Initial kernel scaffold; baseline (speedup 1.0000x reference)
import jax
import jax.numpy as jnp
from jax.experimental import pallas as pl


def kernel(inputs, embeddings):
    raise NotImplementedError("write your pallas kernel here")


# speedup vs baseline: 1.2360x; 1.2360x over previous
import functools

import jax
import jax.numpy as jnp
from jax import lax
from jax.experimental import pallas as pl
from jax.experimental.pallas import tpu as pltpu
from jax.experimental.pallas import tpu_sc as plsc

NUM_EMB = 8192
DIM = 32
N_TOK = 16384
BM = 512
NB = N_TOK // BM
BN = 2048
HALF = NUM_EMB // 2


def _half_argmin(x, x2, et_ref, e2_ref, lo, hi):
    best_val = jnp.full((BM, 1), jnp.inf, dtype=jnp.float32)
    best_idx = jnp.zeros((BM, 1), dtype=jnp.int32)
    for k0 in range(lo, hi, BN):
        et = et_ref[:, k0:k0 + BN]
        mm = lax.dot_general(x, et, (((1,), (0,)), ((), ())),
                             preferred_element_type=jnp.float32)
        d = x2 - 2.0 * mm + e2_ref[:, k0:k0 + BN]
        dmin = jnp.min(d, axis=1, keepdims=True)
        cols = lax.broadcasted_iota(jnp.int32, d.shape, 1) + k0
        lidx = jnp.min(jnp.where(d == dmin, cols, NUM_EMB), axis=1,
                       keepdims=True)
        upd = dmin < best_val
        best_idx = jnp.where(upd, lidx, best_idx)
        best_val = jnp.where(upd, dmin, best_val)
    return best_val, best_idx


def _argmin_body(x_ref, et_ref, x2_ref, e2_ref, idx_ref, dsum_ref):
    x = x_ref[...]
    x2 = x2_ref[...]
    val_a, idx_a = _half_argmin(x, x2, et_ref, e2_ref, 0, HALF)
    val_b, idx_b = _half_argmin(x, x2, et_ref, e2_ref, HALF, NUM_EMB)

    pick_a = val_a.astype(jnp.bfloat16).astype(jnp.float32) <= val_b
    best_idx = jnp.where(pick_a, idx_a, idx_b)
    best_val = jnp.where(pick_a, val_a, val_b)

    idx_ref[...] = best_idx.reshape(1, BM, 1)

    @pl.when(pl.program_id(0) == 0)
    def _init():
        dsum_ref[...] = jnp.zeros_like(dsum_ref)

    dsum_ref[...] += jnp.sum(best_val).reshape(1, 1)


def _run_argmin(flat, e_t, x2, e2_row):
    return pl.pallas_call(
        _argmin_body,
        grid=(NB,),
        in_specs=[
            pl.BlockSpec((BM, DIM), lambda i: (i, 0)),
            pl.BlockSpec((DIM, NUM_EMB), lambda i: (0, 0)),
            pl.BlockSpec((BM, 1), lambda i: (i, 0)),
            pl.BlockSpec((1, NUM_EMB), lambda i: (0, 0)),
        ],
        out_specs=[
            pl.BlockSpec((1, BM, 1), lambda i: (i, 0, 0)),
            pl.BlockSpec((1, 1), lambda i: (0, 0)),
        ],
        out_shape=[
            jax.ShapeDtypeStruct((NB, BM, 1), jnp.int32),
            jax.ShapeDtypeStruct((1, 1), jnp.float32),
        ],
    )(flat, e_t, x2, e2_row)


_NC = 2
_NS = 16
_NW = _NC * _NS
_BPW = N_TOK // _NW
_CHUNK = 128
_NCH = _BPW // _CHUNK


def _gather_body(table_hbm, idx_hbm, out_hbm, idx_v, rows_v, sem):
    wid = lax.axis_index("s") * _NC + lax.axis_index("c")
    pltpu.sync_copy(idx_hbm.at[wid], idx_v)
    copies = []
    for k in range(_NCH):
        copies.append(pltpu.async_copy(
            table_hbm.at[idx_v.at[k]],
            rows_v.at[pl.ds(k * _CHUNK, _CHUNK)], sem))
    for c in copies:
        c.wait()
    pltpu.sync_copy(rows_v, out_hbm.at[pl.ds(wid * _BPW, _BPW)])


def _run_gather(embeddings, idx):
    mesh = plsc.VectorSubcoreMesh(core_axis_name="c", subcore_axis_name="s")
    gk = functools.partial(
        pl.kernel,
        mesh=mesh,
        out_type=jax.ShapeDtypeStruct((N_TOK, DIM), jnp.float32),
        scratch_types=[
            pltpu.VMEM((_NCH, _CHUNK), jnp.int32),
            pltpu.VMEM((_BPW, DIM), jnp.float32),
            pltpu.SemaphoreType.DMA,
        ],
        compiler_params=pltpu.CompilerParams(use_tc_tiling_on_sc=False),
    )(_gather_body)
    return gk(embeddings, idx.reshape(_NW, _NCH, _CHUNK))


def kernel(inputs, embeddings):
    flat = inputs.reshape(-1, DIM)
    x2 = jnp.sum(flat ** 2, axis=1, keepdims=True)
    e2 = jnp.sum(embeddings ** 2, axis=1)
    idx3, dsum = _run_argmin(flat, embeddings.T, x2, e2.reshape(1, NUM_EMB))
    idx = idx3.reshape(N_TOK)
    quantized = _run_gather(embeddings, idx).reshape(inputs.shape)
    loss = dsum[0, 0] * jnp.float32(1.25 / (N_TOK * DIM))
    quantized_st = inputs + lax.stop_gradient(quantized - inputs)
    return (quantized_st, loss)


# speedup vs baseline: 1.2797x; 1.0353x over previous
import functools

import jax
import jax.numpy as jnp
from jax import lax
from jax.experimental import pallas as pl
from jax.experimental.pallas import tpu as pltpu
from jax.experimental.pallas import tpu_sc as plsc

NUM_EMB = 8192
DIM = 32
N_TOK = 16384
BM = 1024
NB = N_TOK // BM
BN = 2048
HALF = NUM_EMB // 2


def _half_argmin(x, x2, et_ref, e2_ref, lo, hi):
    best_val = jnp.full((BM, 1), jnp.inf, dtype=jnp.float32)
    best_idx = jnp.zeros((BM, 1), dtype=jnp.int32)
    for k0 in range(lo, hi, BN):
        et = et_ref[:, k0:k0 + BN]
        mm = lax.dot_general(x, et, (((1,), (0,)), ((), ())),
                             preferred_element_type=jnp.float32)
        d = x2 - 2.0 * mm + e2_ref[:, k0:k0 + BN]
        dmin = jnp.min(d, axis=1, keepdims=True)
        cols = lax.broadcasted_iota(jnp.int32, d.shape, 1) + k0
        lidx = jnp.min(jnp.where(d == dmin, cols, NUM_EMB), axis=1,
                       keepdims=True)
        upd = dmin < best_val
        best_idx = jnp.where(upd, lidx, best_idx)
        best_val = jnp.where(upd, dmin, best_val)
    return best_val, best_idx


def _argmin_body(x_ref, et_ref, x2_ref, e2_ref, idx_ref, dsum_ref):
    x = x_ref[...]
    x2 = x2_ref[...]
    val_a, idx_a = _half_argmin(x, x2, et_ref, e2_ref, 0, HALF)
    val_b, idx_b = _half_argmin(x, x2, et_ref, e2_ref, HALF, NUM_EMB)

    pick_a = val_a.astype(jnp.bfloat16).astype(jnp.float32) <= val_b
    best_idx = jnp.where(pick_a, idx_a, idx_b)
    best_val = jnp.where(pick_a, val_a, val_b)

    idx_ref[...] = best_idx.reshape(1, BM, 1)

    @pl.when(pl.program_id(0) == 0)
    def _init():
        dsum_ref[...] = jnp.zeros_like(dsum_ref)

    dsum_ref[...] += jnp.sum(best_val).reshape(1, 1)


def _run_argmin(flat, e_t, x2, e2_row):
    return pl.pallas_call(
        _argmin_body,
        grid=(NB,),
        in_specs=[
            pl.BlockSpec((BM, DIM), lambda i: (i, 0)),
            pl.BlockSpec((DIM, NUM_EMB), lambda i: (0, 0)),
            pl.BlockSpec((BM, 1), lambda i: (i, 0)),
            pl.BlockSpec((1, NUM_EMB), lambda i: (0, 0)),
        ],
        out_specs=[
            pl.BlockSpec((1, BM, 1), lambda i: (i, 0, 0)),
            pl.BlockSpec((1, 1), lambda i: (0, 0)),
        ],
        out_shape=[
            jax.ShapeDtypeStruct((NB, BM, 1), jnp.int32),
            jax.ShapeDtypeStruct((1, 1), jnp.float32),
        ],
    )(flat, e_t, x2, e2_row)


_NC = 2
_NS = 16
_NW = _NC * _NS
_BPW = N_TOK // _NW
_CHUNK = 128
_NCH = _BPW // _CHUNK


def _gather_body(table_hbm, idx_hbm, out_hbm, idx_v, rows_v, sem):
    wid = lax.axis_index("s") * _NC + lax.axis_index("c")
    pltpu.sync_copy(idx_hbm.at[wid], idx_v)
    copies = []
    for k in range(_NCH):
        copies.append(pltpu.async_copy(
            table_hbm.at[idx_v.at[k]],
            rows_v.at[pl.ds(k * _CHUNK, _CHUNK)], sem))
    for c in copies:
        c.wait()
    pltpu.sync_copy(rows_v, out_hbm.at[pl.ds(wid * _BPW, _BPW)])


def _run_gather(embeddings, idx):
    mesh = plsc.VectorSubcoreMesh(core_axis_name="c", subcore_axis_name="s")
    gk = functools.partial(
        pl.kernel,
        mesh=mesh,
        out_type=jax.ShapeDtypeStruct((N_TOK, DIM), jnp.float32),
        scratch_types=[
            pltpu.VMEM((_NCH, _CHUNK), jnp.int32),
            pltpu.VMEM((_BPW, DIM), jnp.float32),
            pltpu.SemaphoreType.DMA,
        ],
        compiler_params=pltpu.CompilerParams(use_tc_tiling_on_sc=False),
    )(_gather_body)
    return gk(embeddings, idx.reshape(_NW, _NCH, _CHUNK))


def kernel(inputs, embeddings):
    flat = inputs.reshape(-1, DIM)
    x2 = jnp.sum(flat ** 2, axis=1, keepdims=True)
    e2 = jnp.sum(embeddings ** 2, axis=1)
    idx3, dsum = _run_argmin(flat, embeddings.T, x2, e2.reshape(1, NUM_EMB))
    idx = idx3.reshape(N_TOK)
    quantized = _run_gather(embeddings, idx).reshape(inputs.shape)
    loss = dsum[0, 0] * jnp.float32(1.25 / (N_TOK * DIM))
    quantized_st = inputs + lax.stop_gradient(quantized - inputs)
    return (quantized_st, loss)
